# baseline (speedup 1.0000x reference)
import jax
import jax.numpy as jnp
from jax import lax
from jax.experimental import pallas as pl
from jax.experimental.pallas import tpu as pltpu
from jax.experimental.pallas import tpu_sc as plsc

B, C, N, K = 4, 64, 4096, 16
HID, OUT, G = 64, 128, 4

T1 = 256
T2 = 512
NEG = -3.0e38

EDGES = B * N * K
NC, NS = 2, 16
NW = NC * NS
E_PER_W = EDGES // NW
CH = 128
NCHUNK = E_PER_W // CH
NBUF = 4


NGRP = 8
GW = N // NGRP
GBITS = GW - 1
NCAND = 9
IMIN = -2147483648


def _knn_proj_body(f_ref, ft_ref, w1t_ref, b1_ref, idx_ref, u_ref, v_ref):
    b = pl.program_id(0)
    i = pl.program_id(1)
    f = f_ref[0]
    ftc = ft_ref[0]
    cc = (((0,), (0,)), ((), ()))
    inner = lax.dot_general(ftc, f, cc,
                            preferred_element_type=jnp.float32)
    xx = jnp.sum(f * f, axis=0, keepdims=True)
    xxt = jnp.sum(ftc * ftc, axis=0, keepdims=True).T
    dist = 2.0 * inner - xx - xxt

    lane = lax.broadcasted_iota(jnp.int32, (T1, N), 1)
    row = lax.broadcasted_iota(jnp.int32, (T1, 1), 0) + i * T1
    dm = jnp.where(lane == row, NEG, dist)
    d2 = jnp.max(dm, axis=1, keepdims=True)
    dp = (dm - d2) - 1e-12
    bits = lax.bitcast_convert_type(dp, jnp.int32)
    key = lax.bitcast_convert_type((bits & ~GBITS) | (lane & GBITS),
                                   jnp.float32)

    cand_k, cand_i = [], []
    for g in range(NGRP):
        kg = key[:, g * GW:(g + 1) * GW]
        for k in range(NCAND):
            m = jnp.max(kg, axis=1, keepdims=True)
            cand_k.append(m)
            loc = lax.bitcast_convert_type(m, jnp.int32) & GBITS
            cand_i.append((loc + g * GW).astype(jnp.float32))
            if k < NCAND - 1:
                kg = jnp.where(kg == m, NEG, kg)

    base = b * N
    ck = jnp.concatenate(cand_k, axis=1)
    cx = jnp.concatenate(cand_i, axis=1)
    cols = [(row + base).astype(jnp.float32)]
    for k in range(K - 1):
        m = jnp.max(ck, axis=1, keepdims=True)
        eq = ck == m
        cols.append(jnp.max(jnp.where(eq, cx, -1.0), axis=1, keepdims=True)
                    + base)
        if k < K - 2:
            ck = jnp.where(eq, NEG, ck)
    idx_ref[...] = jnp.concatenate(cols, axis=1).astype(jnp.int32)

    w1t = w1t_ref[...]
    w1at = w1t[:C]
    w1bt = w1t[C:]
    u_ref[...] = lax.dot_general(ftc, w1at, cc,
                                 preferred_element_type=jnp.float32)
    v_ref[...] = (lax.dot_general(ftc, w1bt - w1at, cc,
                                  preferred_element_type=jnp.float32)
                  + b1_ref[...])


def _gather_body(u_hbm, idx_hbm, out_hbm, idx_v, bufs, sems):
    wid = lax.axis_index("s") * NC + lax.axis_index("c")
    base = wid * E_PER_W
    pltpu.sync_copy(idx_hbm.at[wid], idx_v)

    for j in range(NBUF):
        pltpu.async_copy(u_hbm.at[idx_v.at[j]], bufs[j], sems[j])

    @pl.loop(0, NCHUNK, step=NBUF)
    def _(i):
        for j in range(NBUF):
            pltpu.make_async_copy(u_hbm.at[idx_v.at[j]], bufs[j], sems[j]).wait()
            pltpu.sync_copy(bufs[j], out_hbm.at[pl.ds(base + (i + j) * CH, CH)])
            nxt = i + j + NBUF

            @pl.when(nxt < NCHUNK)
            def _():
                pltpu.async_copy(u_hbm.at[idx_v.at[nxt]], bufs[j], sems[j])


def _conv_body(g_ref, v_ref, w2t_ref, b2_ref, out_ref):
    g = g_ref[...]
    v = v_ref[...]
    h = jnp.maximum(g.reshape(T2, K, HID) + v[:, None, :], 0.0)
    o = jnp.dot(h.reshape(T2 * K, HID), w2t_ref[...],
                preferred_element_type=jnp.float32)
    out_ref[...] = jnp.max(o.reshape(T2, K, OUT), axis=1) + b2_ref[...]


def _knn_proj(feats, featsT, w1t, b1):
    nrow = N // T1
    return pl.pallas_call(
        _knn_proj_body,
        grid=(B, nrow),
        in_specs=[
            pl.BlockSpec((1, C, N), lambda b, i: (b, 0, 0)),
            pl.BlockSpec((1, C, T1), lambda b, i: (b, 0, i)),
            pl.BlockSpec((2 * C, HID), lambda b, i: (0, 0)),
            pl.BlockSpec((1, HID), lambda b, i: (0, 0)),
        ],
        out_specs=[
            pl.BlockSpec((T1, K), lambda b, i: (b * nrow + i, 0)),
            pl.BlockSpec((T1, HID), lambda b, i: (b * nrow + i, 0)),
            pl.BlockSpec((T1, HID), lambda b, i: (b * nrow + i, 0)),
        ],
        out_shape=[
            jax.ShapeDtypeStruct((B * N, K), jnp.int32),
            jax.ShapeDtypeStruct((B * N, HID), jnp.float32),
            jax.ShapeDtypeStruct((B * N, HID), jnp.float32),
        ],
    )(feats, featsT, w1t, b1)


def _sc_gather(u, idx3):
    mesh = plsc.VectorSubcoreMesh(core_axis_name="c", subcore_axis_name="s")
    kfn = pl.kernel(
        _gather_body,
        out_type=jax.ShapeDtypeStruct((EDGES, HID), jnp.float32),
        mesh=mesh,
        scratch_types=[
            pltpu.VMEM((NCHUNK, CH), jnp.int32),
            [pltpu.VMEM((CH, HID), jnp.float32) for _ in range(NBUF)],
            [pltpu.SemaphoreType.DMA for _ in range(NBUF)],
        ],
        compiler_params=pltpu.CompilerParams(use_tc_tiling_on_sc=False),
    )
    return kfn(u, idx3)


def _conv(g, v, w2t, b2):
    npt = B * N // T2
    return pl.pallas_call(
        _conv_body,
        grid=(npt,),
        in_specs=[
            pl.BlockSpec((T2 * K, HID), lambda i: (i, 0)),
            pl.BlockSpec((T2, HID), lambda i: (i, 0)),
            pl.BlockSpec((HID, OUT), lambda i: (0, 0)),
            pl.BlockSpec((1, OUT), lambda i: (0, 0)),
        ],
        out_specs=pl.BlockSpec((T2, OUT), lambda i: (i, 0)),
        out_shape=jax.ShapeDtypeStruct((B * N, OUT), jnp.float32),
    )(g, v, w2t, b2)


def kernel(feats, W1, b1, W2, b2):
    idx, u, v = _knn_proj(feats, feats, W1.T, b1.reshape(1, HID))
    idx3 = idx.reshape(NW, NCHUNK, CH)
    g = _sc_gather(u, idx3)
    out = _conv(g, v, W2.T, b2.reshape(1, OUT))
    return out.reshape(B, OUT // G, N, G)


# speedup vs baseline: 1.3952x; 1.3952x over previous
import jax
import jax.numpy as jnp
from jax import lax
from jax.experimental import pallas as pl
from jax.experimental.pallas import tpu as pltpu
from jax.experimental.pallas import tpu_sc as plsc

B, C, N, K = 4, 64, 4096, 16
HID, OUT, G = 64, 128, 4

T1 = 256
T2 = 512
NEG = -3.0e38

EDGES = B * N * K
NC, NS = 2, 16
NW = NC * NS
E_PER_W = EDGES // NW
CH = 128
NCHUNK = E_PER_W // CH
NBUF = 4


NGRP = 8
GW = N // NGRP
GBITS = GW - 1
NCAND = 9
IMIN = -2147483648


def _knn_proj_body(f_ref, ft_ref, w1t_ref, b1_ref, idx_ref, u_ref, v_ref):
    b = pl.program_id(0)
    i = pl.program_id(1)
    f = f_ref[0]
    ftc = ft_ref[0]
    cc = (((0,), (0,)), ((), ()))
    inner = lax.dot_general(ftc, f, cc,
                            preferred_element_type=jnp.float32)
    xx = jnp.sum(f * f, axis=0, keepdims=True)
    xxt = jnp.sum(ftc * ftc, axis=0, keepdims=True).T
    dist = 2.0 * inner - xx - xxt

    lane = lax.broadcasted_iota(jnp.int32, (T1, N), 1)
    row = lax.broadcasted_iota(jnp.int32, (T1, 1), 0) + i * T1
    dm = jnp.where(lane == row, NEG, dist)
    d2 = jnp.max(dm, axis=1, keepdims=True)
    dp = (dm - d2) - 1e-12
    bits = lax.bitcast_convert_type(dp, jnp.int32)
    key = lax.bitcast_convert_type((bits & ~GBITS) | (lane & GBITS),
                                   jnp.float32)

    cand_k, cand_i = [], []
    for g in range(NGRP):
        kg = key[:, g * GW:(g + 1) * GW]
        for k in range(NCAND):
            m = jnp.max(kg, axis=1, keepdims=True)
            cand_k.append(m)
            loc = lax.bitcast_convert_type(m, jnp.int32) & GBITS
            cand_i.append((loc + g * GW).astype(jnp.float32))
            if k < NCAND - 1:
                kg = jnp.where(kg == m, NEG, kg)

    base = b * N
    ck = jnp.concatenate(cand_k, axis=1)
    cx = jnp.concatenate(cand_i, axis=1)
    cols = [(row + base).astype(jnp.float32)]
    for k in range(K - 1):
        m = jnp.max(ck, axis=1, keepdims=True)
        eq = ck == m
        cols.append(jnp.max(jnp.where(eq, cx, -1.0), axis=1, keepdims=True)
                    + base)
        if k < K - 2:
            ck = jnp.where(eq, NEG, ck)
    idx_ref[...] = jnp.concatenate(cols, axis=1).astype(jnp.int32)

    w1t = w1t_ref[...]
    w1at = w1t[:C]
    w1bt = w1t[C:]
    u_ref[...] = lax.dot_general(ftc, w1at, cc,
                                 preferred_element_type=jnp.float32)
    v_ref[...] = (lax.dot_general(ftc, w1bt - w1at, cc,
                                  preferred_element_type=jnp.float32)
                  + b1_ref[...])


def _gather_body(u_hbm, idx_hbm, out_hbm, idx_v, bufs, sems):
    wid = lax.axis_index("s") * NC + lax.axis_index("c")
    base = wid * E_PER_W
    pltpu.sync_copy(idx_hbm.at[wid], idx_v)

    for j in range(NBUF):
        pltpu.async_copy(u_hbm.at[idx_v.at[j]], bufs[j], sems[j])

    @pl.loop(0, NCHUNK, step=NBUF)
    def _(i):
        for j in range(NBUF):
            pltpu.make_async_copy(u_hbm.at[idx_v.at[j]], bufs[j], sems[j]).wait()
            pltpu.sync_copy(bufs[j], out_hbm.at[pl.ds(base + (i + j) * CH, CH)])
            nxt = i + j + NBUF

            @pl.when(nxt < NCHUNK)
            def _():
                pltpu.async_copy(u_hbm.at[idx_v.at[nxt]], bufs[j], sems[j])


def _conv_body(g_ref, v_ref, w2t_ref, b2_ref, out_ref):
    g = g_ref[...]
    v = v_ref[...]
    h = jnp.maximum(g.reshape(T2, K, HID) + v[:, None, :], 0.0)
    o = jnp.dot(h.reshape(T2 * K, HID), w2t_ref[...],
                preferred_element_type=jnp.float32)
    out_ref[...] = jnp.max(o.reshape(T2, K, OUT), axis=1) + b2_ref[...]


def _knn_proj(feats, featsT, w1t, b1):
    nrow = N // T1
    return pl.pallas_call(
        _knn_proj_body,
        grid=(B, nrow),
        in_specs=[
            pl.BlockSpec((1, C, N), lambda b, i: (b, 0, 0)),
            pl.BlockSpec((1, C, T1), lambda b, i: (b, 0, i)),
            pl.BlockSpec((2 * C, HID), lambda b, i: (0, 0)),
            pl.BlockSpec((1, HID), lambda b, i: (0, 0)),
        ],
        out_specs=[
            pl.BlockSpec((T1, K), lambda b, i: (b * nrow + i, 0)),
            pl.BlockSpec((T1, HID), lambda b, i: (b * nrow + i, 0)),
            pl.BlockSpec((T1, HID), lambda b, i: (b * nrow + i, 0)),
        ],
        out_shape=[
            jax.ShapeDtypeStruct((B * N, K), jnp.int32),
            jax.ShapeDtypeStruct((B * N, HID), jnp.float32),
            jax.ShapeDtypeStruct((B * N, HID), jnp.float32),
        ],
    )(feats, featsT, w1t, b1)


def _sc_gather(u, idx3):
    mesh = plsc.VectorSubcoreMesh(core_axis_name="c", subcore_axis_name="s")
    kfn = pl.kernel(
        _gather_body,
        out_type=jax.ShapeDtypeStruct((EDGES, HID), jnp.float32),
        mesh=mesh,
        scratch_types=[
            pltpu.VMEM((NCHUNK, CH), jnp.int32),
            [pltpu.VMEM((CH, HID), jnp.float32) for _ in range(NBUF)],
            [pltpu.SemaphoreType.DMA for _ in range(NBUF)],
        ],
        compiler_params=pltpu.CompilerParams(use_tc_tiling_on_sc=False),
    )
    return kfn(u, idx3)


def _conv(g, v, w2t, b2):
    npt = B * N // T2
    return pl.pallas_call(
        _conv_body,
        grid=(npt,),
        in_specs=[
            pl.BlockSpec((T2 * K, HID), lambda i: (i, 0)),
            pl.BlockSpec((T2, HID), lambda i: (i, 0)),
            pl.BlockSpec((HID, OUT), lambda i: (0, 0)),
            pl.BlockSpec((1, OUT), lambda i: (0, 0)),
        ],
        out_specs=pl.BlockSpec((T2, OUT), lambda i: (i, 0)),
        out_shape=jax.ShapeDtypeStruct((B * N, OUT), jnp.float32),
    )(g, v, w2t, b2)


def kernel(feats, W1, b1, W2, b2):
    idx, u, v = _knn_proj(feats, feats, W1.T, b1.reshape(1, HID))
    idx3 = idx.reshape(NW, NCHUNK, CH)
    g = _sc_gather(u, idx3)
    out = _conv(g, v, W2.T, b2.reshape(1, OUT))
    return out.reshape(B, N, OUT // G, G).transpose(0, 2, 1, 3)


# speedup vs baseline: 1.6084x; 1.1528x over previous
import jax
import jax.numpy as jnp
from jax import lax
from jax.experimental import pallas as pl
from jax.experimental.pallas import tpu as pltpu
from jax.experimental.pallas import tpu_sc as plsc

B, C, N, K = 4, 64, 4096, 16
HID, OUT, G = 64, 128, 4

T1 = 256
T2 = 512
NEG = -3.0e38

EDGES = B * N * K
NC, NS = 2, 16
NW = NC * NS
E_PER_W = EDGES // NW
CH = 128
NCHUNK = E_PER_W // CH
NBUF = 4


NGRP = 8
GW = N // NGRP
GBITS = GW - 1
NCAND = 9
IMIN = -2147483648


def _knn_proj_body(f_ref, ft_ref, w1t_ref, b1_ref, idx_ref, u_ref, v_ref):
    b = pl.program_id(0)
    i = pl.program_id(1)
    f = f_ref[0]
    ftc = ft_ref[0]
    cc = (((0,), (0,)), ((), ()))
    inner = lax.dot_general(ftc, f, cc,
                            preferred_element_type=jnp.float32)
    xx = jnp.sum(f * f, axis=0, keepdims=True)
    xxt = jnp.sum(ftc * ftc, axis=0, keepdims=True).T
    dist = 2.0 * inner - xx - xxt

    lane = lax.broadcasted_iota(jnp.int32, (T1, N), 1)
    row = lax.broadcasted_iota(jnp.int32, (T1, 1), 0) + i * T1
    dm = jnp.where(lane == row, NEG, dist)
    d2 = jnp.max(dm, axis=1, keepdims=True)
    dp = (dm - d2) - 1e-12
    bits = lax.bitcast_convert_type(dp, jnp.int32)
    key = lax.bitcast_convert_type((bits & ~GBITS) | (lane & GBITS),
                                   jnp.float32)

    cand_k, cand_i = [], []
    for g in range(NGRP):
        kg = key[:, g * GW:(g + 1) * GW]
        for k in range(NCAND):
            m = jnp.max(kg, axis=1, keepdims=True)
            cand_k.append(m)
            loc = lax.bitcast_convert_type(m, jnp.int32) & GBITS
            cand_i.append((loc + g * GW).astype(jnp.float32))
            if k < NCAND - 1:
                kg = jnp.where(kg == m, NEG, kg)

    base = b * N
    ck = jnp.concatenate(cand_k, axis=1)
    cx = jnp.concatenate(cand_i, axis=1)
    cols = [(row + base).astype(jnp.float32)]
    for k in range(K - 1):
        m = jnp.max(ck, axis=1, keepdims=True)
        eq = ck == m
        cols.append(jnp.max(jnp.where(eq, cx, -1.0), axis=1, keepdims=True)
                    + base)
        if k < K - 2:
            ck = jnp.where(eq, NEG, ck)
    idx_ref[...] = jnp.concatenate(cols, axis=1).astype(jnp.int32)

    w1t = w1t_ref[...]
    w1at = w1t[:C]
    w1bt = w1t[C:]
    u_ref[...] = lax.dot_general(ftc, w1at, cc,
                                 preferred_element_type=jnp.float32)
    v_ref[...] = (lax.dot_general(ftc, w1bt - w1at, cc,
                                  preferred_element_type=jnp.float32)
                  + b1_ref[...])


def _gather_body(u_hbm, idx_hbm, out_hbm, idx_v, bufs, sems):
    wid = lax.axis_index("s") * NC + lax.axis_index("c")
    base = wid * E_PER_W
    pltpu.sync_copy(idx_hbm.at[wid], idx_v)

    for j in range(NBUF):
        pltpu.async_copy(u_hbm.at[idx_v.at[j]], bufs[j], sems[j])

    @pl.loop(0, NCHUNK, step=NBUF)
    def _(i):
        for j in range(NBUF):
            pltpu.make_async_copy(u_hbm.at[idx_v.at[j]], bufs[j], sems[j]).wait()
            pltpu.sync_copy(bufs[j], out_hbm.at[pl.ds(base + (i + j) * CH, CH)])
            nxt = i + j + NBUF

            @pl.when(nxt < NCHUNK)
            def _():
                pltpu.async_copy(u_hbm.at[idx_v.at[nxt]], bufs[j], sems[j])


def _conv_body(g_ref, v_ref, w2t_ref, b2_ref, out_ref):
    g2 = g_ref[...]
    v = v_ref[...]
    kh = K // 2
    vr = jnp.broadcast_to(v[:, None, :], (T2, kh, HID)).reshape(T2 * kh, HID)
    out = None
    for h in range(2):
        he = jnp.maximum(g2[:, h * HID:(h + 1) * HID] + vr, 0.0)
        o = jnp.dot(he, w2t_ref[...],
                    preferred_element_type=jnp.float32)
        o = jnp.max(o.reshape(T2, kh, OUT), axis=1)
        out = o if out is None else jnp.maximum(out, o)
    out_ref[0] = out.T + b2_ref[...]


def _knn_proj(feats, featsT, w1t, b1):
    nrow = N // T1
    return pl.pallas_call(
        _knn_proj_body,
        grid=(B, nrow),
        in_specs=[
            pl.BlockSpec((1, C, N), lambda b, i: (b, 0, 0)),
            pl.BlockSpec((1, C, T1), lambda b, i: (b, 0, i)),
            pl.BlockSpec((2 * C, HID), lambda b, i: (0, 0)),
            pl.BlockSpec((1, HID), lambda b, i: (0, 0)),
        ],
        out_specs=[
            pl.BlockSpec((T1, K), lambda b, i: (b * nrow + i, 0)),
            pl.BlockSpec((T1, HID), lambda b, i: (b * nrow + i, 0)),
            pl.BlockSpec((T1, HID), lambda b, i: (b * nrow + i, 0)),
        ],
        out_shape=[
            jax.ShapeDtypeStruct((B * N, K), jnp.int32),
            jax.ShapeDtypeStruct((B * N, HID), jnp.float32),
            jax.ShapeDtypeStruct((B * N, HID), jnp.float32),
        ],
    )(feats, featsT, w1t, b1)


def _sc_gather(u, idx3):
    mesh = plsc.VectorSubcoreMesh(core_axis_name="c", subcore_axis_name="s")
    kfn = pl.kernel(
        _gather_body,
        out_type=jax.ShapeDtypeStruct((EDGES, HID), jnp.float32),
        mesh=mesh,
        scratch_types=[
            pltpu.VMEM((NCHUNK, CH), jnp.int32),
            [pltpu.VMEM((CH, HID), jnp.float32) for _ in range(NBUF)],
            [pltpu.SemaphoreType.DMA for _ in range(NBUF)],
        ],
        compiler_params=pltpu.CompilerParams(use_tc_tiling_on_sc=False),
    )
    return kfn(u, idx3)


def _conv(g, v, w2, b2):
    npt = N // T2
    return pl.pallas_call(
        _conv_body,
        grid=(B, npt),
        in_specs=[
            pl.BlockSpec((T2 * K // 2, 2 * HID), lambda b, i: (b * npt + i, 0)),
            pl.BlockSpec((T2, HID), lambda b, i: (b * npt + i, 0)),
            pl.BlockSpec((HID, OUT), lambda b, i: (0, 0)),
            pl.BlockSpec((OUT, 1), lambda b, i: (0, 0)),
        ],
        out_specs=pl.BlockSpec((1, OUT, T2), lambda b, i: (b, 0, i)),
        out_shape=jax.ShapeDtypeStruct((B, OUT, N), jnp.float32),
    )(g, v, w2, b2)


def kernel(feats, W1, b1, W2, b2):
    idx, u, v = _knn_proj(feats, feats, W1.T, b1.reshape(1, HID))
    idx3 = idx.reshape(NW, NCHUNK, CH)
    g = _sc_gather(u, idx3)
    g2 = g.reshape(EDGES // 2, 2 * HID)
    exp = _conv(g2, v, W2.T, b2.reshape(OUT, 1))
    return exp.reshape(B, OUT // G, G, N).transpose(0, 1, 3, 2)


# speedup vs baseline: 1.7789x; 1.1060x over previous
import jax
import jax.numpy as jnp
from jax import lax
from jax.experimental import pallas as pl
from jax.experimental.pallas import tpu as pltpu
from jax.experimental.pallas import tpu_sc as plsc

B, C, N, K = 4, 64, 4096, 16
HID, OUT, G = 64, 128, 4

T1 = 256
T2 = 512
NEG = -3.0e38

EDGES = B * N * K
NC, NS = 2, 16
NW = NC * NS
E_PER_W = EDGES // NW
CH = 128
NCHUNK = E_PER_W // CH
NBUF = 4


NGRP = 8
GW = N // NGRP
GBITS = GW - 1
NCAND = 8
IMIN = -2147483648


def _knn_proj_body(f_ref, ft_ref, w1t_ref, b1_ref, idx_ref, u_ref, v_ref):
    b = pl.program_id(0)
    i = pl.program_id(1)
    f = f_ref[0]
    ftc = ft_ref[0]
    cc = (((0,), (0,)), ((), ()))
    inner = lax.dot_general(ftc, f, cc,
                            preferred_element_type=jnp.float32)
    xx = jnp.sum(f * f, axis=0, keepdims=True)
    xxt = jnp.sum(ftc * ftc, axis=0, keepdims=True).T
    dist = 2.0 * inner - xx - xxt

    lane = lax.broadcasted_iota(jnp.int32, (T1, N), 1)
    row = lax.broadcasted_iota(jnp.int32, (T1, 1), 0) + i * T1
    dm = jnp.where(lane == row, NEG, dist)
    d2 = jnp.max(dm, axis=1, keepdims=True)
    dp = (dm - d2) - 1e-12
    bits = lax.bitcast_convert_type(dp, jnp.int32)
    key = lax.bitcast_convert_type((bits & ~GBITS) | (lane & GBITS),
                                   jnp.float32)

    cand_k = []
    for g in range(NGRP):
        kg = key[:, g * GW:(g + 1) * GW]
        for k in range(NCAND):
            m = jnp.max(kg, axis=1, keepdims=True)
            cand_k.append(m)
            if k < NCAND - 1:
                kg = jnp.where(kg == m, NEG, kg)

    base = b * N
    ck = jnp.concatenate(cand_k, axis=1)
    ci = lax.broadcasted_iota(jnp.int32, (T1, NGRP * NCAND), 1)
    cx = ((ci >> 3) * GW).astype(jnp.float32)
    cols = [(row + base).astype(jnp.float32)]
    for k in range(K - 1):
        m = jnp.max(ck, axis=1, keepdims=True)
        eq = ck == m
        goff = jnp.max(jnp.where(eq, cx, -1.0), axis=1, keepdims=True)
        loc = lax.bitcast_convert_type(m, jnp.int32) & GBITS
        cols.append(goff + (loc + base).astype(jnp.float32))
        if k < K - 2:
            ck = jnp.where(eq, NEG, ck)
    idx_ref[...] = jnp.concatenate(cols, axis=1).astype(jnp.int32)

    w1t = w1t_ref[...]
    w1at = w1t[:C]
    w1bt = w1t[C:]
    u_ref[...] = lax.dot_general(ftc, w1at, cc,
                                 preferred_element_type=jnp.float32)
    v_ref[...] = (lax.dot_general(ftc, w1bt - w1at, cc,
                                  preferred_element_type=jnp.float32)
                  + b1_ref[...])


def _gather_body(u_hbm, idx_hbm, out_hbm, idx_v, bufs, sems):
    wid = lax.axis_index("s") * NC + lax.axis_index("c")
    base = wid * E_PER_W
    pltpu.sync_copy(idx_hbm.at[wid], idx_v)

    for j in range(NBUF):
        pltpu.async_copy(u_hbm.at[idx_v.at[j]], bufs[j], sems[j])

    @pl.loop(0, NCHUNK, step=NBUF)
    def _(i):
        for j in range(NBUF):
            pltpu.make_async_copy(u_hbm.at[idx_v.at[j]], bufs[j], sems[j]).wait()
            pltpu.sync_copy(bufs[j], out_hbm.at[pl.ds(base + (i + j) * CH, CH)])
            nxt = i + j + NBUF

            @pl.when(nxt < NCHUNK)
            def _():
                pltpu.async_copy(u_hbm.at[idx_v.at[nxt]], bufs[j], sems[j])


def _conv_body(g_ref, v_ref, w2t_ref, b2_ref, out_ref):
    g2 = g_ref[...]
    v = v_ref[...]
    kh = K // 2
    vr = jnp.broadcast_to(v[:, None, :], (T2, kh, HID)).reshape(T2 * kh, HID)
    out = None
    for h in range(2):
        he = jnp.maximum(g2[:, h * HID:(h + 1) * HID] + vr, 0.0)
        o = jnp.dot(he, w2t_ref[...],
                    preferred_element_type=jnp.float32)
        o = jnp.max(o.reshape(T2, kh, OUT), axis=1)
        out = o if out is None else jnp.maximum(out, o)
    out_ref[0] = out.T + b2_ref[...]


def _knn_proj(feats, featsT, w1t, b1):
    nrow = N // T1
    return pl.pallas_call(
        _knn_proj_body,
        grid=(B, nrow),
        in_specs=[
            pl.BlockSpec((1, C, N), lambda b, i: (b, 0, 0)),
            pl.BlockSpec((1, C, T1), lambda b, i: (b, 0, i)),
            pl.BlockSpec((2 * C, HID), lambda b, i: (0, 0)),
            pl.BlockSpec((1, HID), lambda b, i: (0, 0)),
        ],
        out_specs=[
            pl.BlockSpec((T1, K), lambda b, i: (b * nrow + i, 0)),
            pl.BlockSpec((T1, HID), lambda b, i: (b * nrow + i, 0)),
            pl.BlockSpec((T1, HID), lambda b, i: (b * nrow + i, 0)),
        ],
        out_shape=[
            jax.ShapeDtypeStruct((B * N, K), jnp.int32),
            jax.ShapeDtypeStruct((B * N, HID), jnp.float32),
            jax.ShapeDtypeStruct((B * N, HID), jnp.float32),
        ],
    )(feats, featsT, w1t, b1)


def _sc_gather(u, idx3):
    mesh = plsc.VectorSubcoreMesh(core_axis_name="c", subcore_axis_name="s")
    kfn = pl.kernel(
        _gather_body,
        out_type=jax.ShapeDtypeStruct((EDGES, HID), jnp.float32),
        mesh=mesh,
        scratch_types=[
            pltpu.VMEM((NCHUNK, CH), jnp.int32),
            [pltpu.VMEM((CH, HID), jnp.float32) for _ in range(NBUF)],
            [pltpu.SemaphoreType.DMA for _ in range(NBUF)],
        ],
        compiler_params=pltpu.CompilerParams(use_tc_tiling_on_sc=False),
    )
    return kfn(u, idx3)


def _conv(g, v, w2, b2):
    npt = N // T2
    return pl.pallas_call(
        _conv_body,
        grid=(B, npt),
        in_specs=[
            pl.BlockSpec((T2 * K // 2, 2 * HID), lambda b, i: (b * npt + i, 0)),
            pl.BlockSpec((T2, HID), lambda b, i: (b * npt + i, 0)),
            pl.BlockSpec((HID, OUT), lambda b, i: (0, 0)),
            pl.BlockSpec((OUT, 1), lambda b, i: (0, 0)),
        ],
        out_specs=pl.BlockSpec((1, OUT, T2), lambda b, i: (b, 0, i)),
        out_shape=jax.ShapeDtypeStruct((B, OUT, N), jnp.float32),
    )(g, v, w2, b2)


def kernel(feats, W1, b1, W2, b2):
    idx, u, v = _knn_proj(feats, feats, W1.T, b1.reshape(1, HID))
    idx3 = idx.reshape(NW, NCHUNK, CH)
    g = _sc_gather(u, idx3)
    g2 = g.reshape(EDGES // 2, 2 * HID)
    exp = _conv(g2, v, W2.T, b2.reshape(OUT, 1))
    return exp.reshape(B, OUT // G, G, N).transpose(0, 1, 3, 2)


# speedup vs baseline: 2.0570x; 1.1564x over previous
import jax
import jax.numpy as jnp
from jax import lax
from jax.experimental import pallas as pl
from jax.experimental.pallas import tpu as pltpu
from jax.experimental.pallas import tpu_sc as plsc

B, C, N, K = 4, 64, 4096, 16
HID, OUT, G = 64, 128, 4

T1 = 512
T2 = 512
NEG = -3.0e38

EDGES = B * N * K
NC, NS = 2, 16
NW = NC * NS
E_PER_W = EDGES // NW
CH = 128
NCHUNK = E_PER_W // CH
NBUF = 4


NGRP = 8
GW = N // NGRP
GBITS = GW - 1
NCAND = 8
IMIN = -2147483648


def _knn_proj_body(f_ref, ft_ref, w1t_ref, b1_ref, idx_ref, u_ref, v_ref):
    b = pl.program_id(0)
    i = pl.program_id(1)
    f = f_ref[0]
    ftc = ft_ref[0]
    cc = (((0,), (0,)), ((), ()))
    inner = lax.dot_general(ftc, f, cc,
                            preferred_element_type=jnp.float32)
    xx = jnp.sum(f * f, axis=0, keepdims=True)
    xxt = jnp.sum(ftc * ftc, axis=0, keepdims=True).T
    dist = 2.0 * inner - xx - xxt

    lane = lax.broadcasted_iota(jnp.int32, (T1, N), 1)
    row = lax.broadcasted_iota(jnp.int32, (T1, 1), 0) + i * T1
    dm = jnp.where(lane == row, NEG, dist)
    d2 = jnp.max(dm, axis=1, keepdims=True)
    dp = (dm - d2) - 1e-12
    bits = lax.bitcast_convert_type(dp, jnp.int32)
    key = lax.bitcast_convert_type((bits & ~GBITS) | (lane & GBITS),
                                   jnp.float32)

    cand_k = []
    for g in range(NGRP):
        kg = key[:, g * GW:(g + 1) * GW]
        for k in range(NCAND):
            m = jnp.max(kg, axis=1, keepdims=True)
            cand_k.append(m)
            if k < NCAND - 1:
                kg = jnp.where(kg == m, NEG, kg)

    base = b * N
    ck = jnp.concatenate(cand_k, axis=1)
    ci = lax.broadcasted_iota(jnp.int32, (T1, NGRP * NCAND), 1)
    cx = ((ci >> 3) * GW).astype(jnp.float32)
    cols = [(row + base).astype(jnp.float32)]
    for k in range(K - 1):
        m = jnp.max(ck, axis=1, keepdims=True)
        eq = ck == m
        goff = jnp.max(jnp.where(eq, cx, -1.0), axis=1, keepdims=True)
        loc = lax.bitcast_convert_type(m, jnp.int32) & GBITS
        cols.append(goff + (loc + base).astype(jnp.float32))
        if k < K - 2:
            ck = jnp.where(eq, NEG, ck)
    idx_ref[...] = jnp.concatenate(cols, axis=1).astype(jnp.int32)

    w1t = w1t_ref[...]
    w1at = w1t[:C]
    w1bt = w1t[C:]
    u_ref[...] = lax.dot_general(ftc, w1at, cc,
                                 preferred_element_type=jnp.float32)
    v_ref[...] = (lax.dot_general(ftc, w1bt - w1at, cc,
                                  preferred_element_type=jnp.float32)
                  + b1_ref[...])


def _gather_body(u_hbm, idx_hbm, out_hbm, idx_v, bufs, sems):
    wid = lax.axis_index("s") * NC + lax.axis_index("c")
    base = wid * E_PER_W
    pltpu.sync_copy(idx_hbm.at[wid], idx_v)

    for j in range(NBUF):
        pltpu.async_copy(u_hbm.at[idx_v.at[j]], bufs[j], sems[j])

    @pl.loop(0, NCHUNK, step=NBUF)
    def _(i):
        for j in range(NBUF):
            pltpu.make_async_copy(u_hbm.at[idx_v.at[j]], bufs[j], sems[j]).wait()
            pltpu.sync_copy(bufs[j], out_hbm.at[pl.ds(base + (i + j) * CH, CH)])
            nxt = i + j + NBUF

            @pl.when(nxt < NCHUNK)
            def _():
                pltpu.async_copy(u_hbm.at[idx_v.at[nxt]], bufs[j], sems[j])


def _conv_body(g_ref, v_ref, w2t_ref, b2_ref, out_ref):
    g2 = g_ref[...]
    v = v_ref[...]
    kh = K // 2
    vr = jnp.broadcast_to(v[:, None, :], (T2, kh, HID)).reshape(T2 * kh, HID)
    out = None
    for h in range(2):
        he = jnp.maximum(g2[:, h * HID:(h + 1) * HID] + vr, 0.0)
        o = jnp.dot(he, w2t_ref[...],
                    preferred_element_type=jnp.float32)
        o = jnp.max(o.reshape(T2, kh, OUT), axis=1)
        out = o if out is None else jnp.maximum(out, o)
    out_ref[0] = out.T + b2_ref[...]


def _knn_proj(feats, featsT, w1t, b1):
    nrow = N // T1
    return pl.pallas_call(
        _knn_proj_body,
        grid=(B, nrow),
        in_specs=[
            pl.BlockSpec((1, C, N), lambda b, i: (b, 0, 0)),
            pl.BlockSpec((1, C, T1), lambda b, i: (b, 0, i)),
            pl.BlockSpec((2 * C, HID), lambda b, i: (0, 0)),
            pl.BlockSpec((1, HID), lambda b, i: (0, 0)),
        ],
        out_specs=[
            pl.BlockSpec((T1, K), lambda b, i: (b * nrow + i, 0)),
            pl.BlockSpec((T1, HID), lambda b, i: (b * nrow + i, 0)),
            pl.BlockSpec((T1, HID), lambda b, i: (b * nrow + i, 0)),
        ],
        out_shape=[
            jax.ShapeDtypeStruct((B * N, K), jnp.int32),
            jax.ShapeDtypeStruct((B * N, HID), jnp.float32),
            jax.ShapeDtypeStruct((B * N, HID), jnp.float32),
        ],
    )(feats, featsT, w1t, b1)


def _sc_gather(u, idx3):
    mesh = plsc.VectorSubcoreMesh(core_axis_name="c", subcore_axis_name="s")
    kfn = pl.kernel(
        _gather_body,
        out_type=jax.ShapeDtypeStruct((EDGES, HID), jnp.float32),
        mesh=mesh,
        scratch_types=[
            pltpu.VMEM((NCHUNK, CH), jnp.int32),
            [pltpu.VMEM((CH, HID), jnp.float32) for _ in range(NBUF)],
            [pltpu.SemaphoreType.DMA for _ in range(NBUF)],
        ],
        compiler_params=pltpu.CompilerParams(use_tc_tiling_on_sc=False),
    )
    return kfn(u, idx3)


def _conv(g, v, w2, b2):
    npt = N // T2
    return pl.pallas_call(
        _conv_body,
        grid=(B, npt),
        in_specs=[
            pl.BlockSpec((T2 * K // 2, 2 * HID), lambda b, i: (b * npt + i, 0)),
            pl.BlockSpec((T2, HID), lambda b, i: (b * npt + i, 0)),
            pl.BlockSpec((HID, OUT), lambda b, i: (0, 0)),
            pl.BlockSpec((OUT, 1), lambda b, i: (0, 0)),
        ],
        out_specs=pl.BlockSpec((1, OUT, T2), lambda b, i: (b, 0, i)),
        out_shape=jax.ShapeDtypeStruct((B, OUT, N), jnp.float32),
    )(g, v, w2, b2)


def kernel(feats, W1, b1, W2, b2):
    idx, u, v = _knn_proj(feats, feats, W1.T, b1.reshape(1, HID))
    idx3 = idx.reshape(NW, NCHUNK, CH)
    g = _sc_gather(u, idx3)
    g2 = g.reshape(EDGES // 2, 2 * HID)
    exp = _conv(g2, v, W2.T, b2.reshape(OUT, 1))
    return exp.reshape(B, OUT // G, G, N).transpose(0, 1, 3, 2)
